# baseline (speedup 1.0000x reference)
import functools

import jax
import jax.numpy as jnp
from jax import lax
from jax.experimental import pallas as pl
from jax.experimental.pallas import tpu as pltpu
from jax.experimental.pallas import tpu_sc as plsc

NUM_TERMS = 1000000
D = 32
KREL = 32
NRELS = 40
B = 16384

NW = 32
TOT = 2 * B
BPW = TOT // NW
NCH = BPW // 128

@functools.cache
def _get_sc_gather():
    mesh = plsc.VectorSubcoreMesh(core_axis_name="c", subcore_axis_name="s")

    @functools.partial(
        pl.kernel,
        mesh=mesh,
        out_type=jax.ShapeDtypeStruct((TOT, D), jnp.float32),
        scratch_types=[
            pltpu.VMEM((NCH, 128), jnp.int32),
            pltpu.VMEM((BPW, D), jnp.float32),
            pltpu.SemaphoreType.DMA,
        ],
        compiler_params=pltpu.CompilerParams(use_tc_tiling_on_sc=False),
    )
    def _sc_gather(table_hbm, idx_hbm, out_hbm, idx_v, rows_v, sem):
        wid = lax.axis_index("s") * 2 + lax.axis_index("c")
        pltpu.sync_copy(idx_hbm.at[wid], idx_v)
        copies = []
        for j in range(NCH):
            copies.append(
                pltpu.async_copy(
                    table_hbm.at[idx_v.at[j]],
                    rows_v.at[pl.ds(j * 128, 128)],
                    sem,
                )
            )
        for c in copies:
            c.wait()
        pltpu.sync_copy(rows_v, out_hbm.at[pl.ds(wid * BPW, BPW)])

    return _sc_gather


BB = 512
NB = B // BB


def _tc_body(l_ref, r_ref, rels_ref, w2_ref, g_ref, rv_ref, b_ref, out_ref):
    lb = l_ref[...]
    rb = r_ref[...]
    t = jnp.dot(lb, w2_ref[...], preferred_element_type=jnp.float32)
    rrep = jnp.concatenate([rb] * KREL, axis=1)
    p = t * rrep
    s = jnp.dot(p, g_ref[...], preferred_element_type=jnp.float32)
    ridx = rels_ref[...]
    onehot = (lax.broadcasted_iota(jnp.int32, (BB, NRELS), 1) == ridx
              ).astype(jnp.float32)
    biascol = jnp.dot(rv_ref[...], b_ref[...],
                      preferred_element_type=jnp.float32)
    energy = (jnp.sum(s * onehot, axis=1, keepdims=True)
              + jnp.dot(onehot, biascol, preferred_element_type=jnp.float32))
    out_ref[...] = energy


def _tc_score(lrows, rrows, rels2d, w2, g, rel_vecs, b2):
    return pl.pallas_call(
        _tc_body,
        grid=(NB,),
        in_specs=[
            pl.BlockSpec((BB, D), lambda i: (i, 0)),
            pl.BlockSpec((BB, D), lambda i: (i, 0)),
            pl.BlockSpec((BB, 1), lambda i: (i, 0)),
            pl.BlockSpec((D, KREL * D), lambda i: (0, 0)),
            pl.BlockSpec((KREL * D, NRELS), lambda i: (0, 0)),
            pl.BlockSpec((NRELS, KREL), lambda i: (0, 0)),
            pl.BlockSpec((KREL, 1), lambda i: (0, 0)),
        ],
        out_specs=pl.BlockSpec((BB, 1), lambda i: (i, 0)),
        out_shape=jax.ShapeDtypeStruct((B, 1), jnp.float32),
    )(lrows, rrows, rels2d, w2, g, rel_vecs, b2)


def kernel(term_vecs, rel_vecs, assoc_W, assoc_b, rels, terms_L, terms_R):
    idx = jnp.concatenate([terms_L, terms_R]).astype(jnp.int32)
    idx = idx.reshape(NW, NCH, 128)
    gathered = _get_sc_gather()(term_vecs, idx)
    lrows = gathered[:B]
    rrows = gathered[B:]
    w2 = assoc_W.transpose(1, 0, 2).reshape(D, KREL * D)
    g = jnp.repeat(rel_vecs.T, D, axis=0)
    b2 = assoc_b.reshape(KREL, 1)
    rels2d = rels.astype(jnp.int32).reshape(B, 1)
    energy = _tc_score(lrows, rrows, rels2d, w2, g, rel_vecs, b2)
    return energy.reshape(B)
